# baseline (speedup 1.0000x reference)
import functools

import jax
import jax.numpy as jnp
from jax.experimental import pallas as pl
from jax.experimental.pallas import tpu as pltpu

_COMMITMENT = 0.25


def _vq_kernel(xn_ref, x_ref, cb_ref, q_ref, perp_ref, loss_ref,
               counts_s, loss_s, *, B, K, C, d, T):
    k = pl.program_id(0)
    b = pl.program_id(1)

    xb = x_ref[0, 0]
    cb = cb_ref[0]
    xn = xn_ref[0, 0]

    cn = jnp.sum(cb * cb, axis=1, keepdims=True)

    mm = jax.lax.dot_general(cb, xb, (((1,), (0,)), ((), ())),
                             preferred_element_type=jnp.float32)
    dist = (xn + cn) + (-2.0 * mm)

    m = jnp.min(dist, axis=0, keepdims=True)
    iota = jax.lax.broadcasted_iota(jnp.int32, (C, T), 0)
    idx = jnp.min(jnp.where(dist == m, iota, C), axis=0, keepdims=True)

    oh = (iota == idx).astype(jnp.float32)
    q = jax.lax.dot_general(cb, oh, (((0,), (0,)), ((), ())),
                            preferred_element_type=jnp.float32,
                            precision=jax.lax.Precision.HIGHEST)

    q_ref[0, 0] = xb + (q - xb)

    ones_t = jnp.ones((1, T), dtype=jnp.float32)
    counts_part = jax.lax.dot_general(ones_t, oh, (((1,), (1,)), ((), ())),
                                      preferred_element_type=jnp.float32)

    @pl.when(b == 0)
    def _():
        counts_s[...] = counts_part

    @pl.when(b != 0)
    def _():
        counts_s[...] = counts_s[...] + counts_part

    diff = xb - q
    part = jnp.sum(diff * diff)

    @pl.when((k == 0) & (b == 0))
    def _():
        loss_s[0, 0] = part

    @pl.when((k != 0) | (b != 0))
    def _():
        loss_s[0, 0] = loss_s[0, 0] + part

    @pl.when(b == B - 1)
    def _():
        p = counts_s[...] * (1.0 / (B * T))
        ent = -jnp.sum(p * jnp.log(p + 1e-8))
        perp_ref[...] = jnp.exp(ent).reshape(1, 1, 1)

    @pl.when((k == K - 1) & (b == B - 1))
    def _():
        n = B * K * d * T
        loss_ref[...] = (loss_s[0, 0] * ((1.0 + _COMMITMENT) / n)).reshape(1, 1)


def kernel(x, codebooks):
    B, D, T = x.shape
    K, C, d = codebooks.shape
    x_chunks = x.reshape(B, K, d, T)
    x_norm = jnp.sum(x_chunks ** 2, axis=2)[:, :, None, :]

    kfn = functools.partial(_vq_kernel, B=B, K=K, C=C, d=d, T=T)
    q, perp, loss = pl.pallas_call(
        kfn,
        grid=(K, B),
        in_specs=[
            pl.BlockSpec((1, 1, 1, T), lambda k, b: (b, k, 0, 0)),
            pl.BlockSpec((1, 1, d, T), lambda k, b: (b, k, 0, 0)),
            pl.BlockSpec((1, C, d), lambda k, b: (k, 0, 0)),
        ],
        out_specs=[
            pl.BlockSpec((1, 1, d, T), lambda k, b: (b, k, 0, 0)),
            pl.BlockSpec((1, 1, 1), lambda k, b: (k, 0, 0)),
            pl.BlockSpec((1, 1), lambda k, b: (0, 0)),
        ],
        out_shape=[
            jax.ShapeDtypeStruct((B, K, d, T), jnp.float32),
            jax.ShapeDtypeStruct((K, 1, 1), jnp.float32),
            jax.ShapeDtypeStruct((1, 1), jnp.float32),
        ],
        scratch_shapes=[
            pltpu.VMEM((1, C), jnp.float32),
            pltpu.SMEM((1, 1), jnp.float32),
        ],
    )(x_norm, x_chunks, codebooks)

    return q.reshape(B, D, T), loss[0, 0], perp.reshape(K)
